# baseline (speedup 1.0000x reference)
import functools

import jax
import jax.numpy as jnp
from jax import lax
from jax.experimental import pallas as pl
from jax.experimental.pallas import tpu as pltpu
from jax.experimental.pallas import tpu_sc as plsc

_B = 4096
_F = 26
_V = 100000
_D = 64
_H0 = 400
_H1 = 400

_NW = 32
_N = _B * _F
_NPW = _N // _NW
_CHUNK = 128
_NCH = _NPW // _CHUNK
_WG = _CHUNK // 16


def _sc_gather(flat_idx, etab, wtab16):
    mesh = plsc.VectorSubcoreMesh(core_axis_name="c", subcore_axis_name="s")

    @functools.partial(
        pl.kernel,
        mesh=mesh,
        out_type=(
            jax.ShapeDtypeStruct((_N, _D), jnp.float32),
            jax.ShapeDtypeStruct((_N,), jnp.float32),
        ),
        scratch_types=[
            pltpu.VMEM((_NCH, _CHUNK), jnp.int32),
            pltpu.VMEM((_CHUNK, _D), jnp.float32),
            pltpu.VMEM((_CHUNK, _D), jnp.float32),
            pltpu.VMEM((_CHUNK, 16), jnp.float32),
            pltpu.VMEM((_CHUNK, 16), jnp.float32),
            pltpu.VMEM((_CHUNK,), jnp.int32),
            pltpu.VMEM((_CHUNK,), jnp.int32),
            pltpu.VMEM((_NPW,), jnp.float32),
            pltpu.SemaphoreType.DMA,
            pltpu.SemaphoreType.DMA,
            pltpu.SemaphoreType.DMA,
            pltpu.SemaphoreType.DMA,
        ],
        compiler_params=pltpu.CompilerParams(use_tc_tiling_on_sc=False,
                                             needs_layout_passes=False),
    )
    def k(idx_hbm, etab_hbm, wtab_hbm, xv_out, xw_out, idx_v,
          rows0, rows1, ww0, ww1, gb0, gb1, xw_buf,
          esem0, esem1, wsem0, wsem1):
        wid = lax.axis_index("s") * 2 + lax.axis_index("c")
        base = wid * _NPW
        pltpu.sync_copy(idx_hbm.at[wid], idx_v)

        def issue(j, rows_v, ww_v, gb_v, esem, wsem):
            for g in range(_WG):
                gb_v[pl.ds(g * 16, 16)] = (
                    idx_v[j, pl.ds(g * 16, 16)] >> 4)
            pltpu.async_copy(etab_hbm.at[idx_v.at[j]], rows_v, esem)
            pltpu.async_copy(wtab_hbm.at[gb_v], ww_v, wsem)

        def drain(j, rows_v, ww_v, gb_v, esem, wsem):
            pltpu.make_async_copy(
                etab_hbm.at[idx_v.at[j]], rows_v, esem).wait()
            pltpu.sync_copy(rows_v, xv_out.at[pl.ds(base + j * _CHUNK,
                                                    _CHUNK), :])
            pltpu.make_async_copy(wtab_hbm.at[gb_v], ww_v, wsem).wait()
            for g in range(_WG):
                sel = idx_v[j, pl.ds(g * 16, 16)] & 15
                rows = lax.broadcasted_iota(jnp.int32, (16,), 0) + g * 16
                vals = plsc.load_gather(ww_v, [rows, sel])
                xw_buf[pl.ds(j * _CHUNK + g * 16, 16)] = vals

        issue(0, rows0, ww0, gb0, esem0, wsem0)

        def body(i, _):
            j0 = 2 * i
            issue(j0 + 1, rows1, ww1, gb1, esem1, wsem1)
            drain(j0, rows0, ww0, gb0, esem0, wsem0)

            @pl.when(j0 + 2 < _NCH)
            def _():
                issue(j0 + 2, rows0, ww0, gb0, esem0, wsem0)

            drain(j0 + 1, rows1, ww1, gb1, esem1, wsem1)
            return ()

        lax.fori_loop(0, _NCH // 2, body, ())
        pltpu.sync_copy(xw_buf, xw_out.at[pl.ds(base, _NPW)])

    return k(flat_idx, etab, wtab16)


def _tc_body(xw_ref, x_ref, w0_ref, b0_ref, w1_ref, b1_ref, w2_ref, bc_ref,
             o_ref):
    x = x_ref[...]
    h = jnp.dot(x, w0_ref[...], preferred_element_type=jnp.float32)
    h = jnp.maximum(h + b0_ref[...], 0.0)
    h = jnp.dot(h, w1_ref[...], preferred_element_type=jnp.float32)
    h = jnp.maximum(h + b1_ref[...], 0.0)
    l = jnp.dot(h, w2_ref[...], preferred_element_type=jnp.float32)[:, 0]
    s = x[:, 0:_D]
    q = s * s
    for f in range(1, _F):
        blk = x[:, f * _D:(f + 1) * _D]
        s = s + blk
        q = q + blk * blk
    p = 0.5 * (jnp.sum(s * s, axis=1) - jnp.sum(q, axis=1))
    xw_sum = jnp.sum(xw_ref[...], axis=1)
    logits = l + xw_sum + bc_ref[0] + p
    o_ref[...] = 1.0 / (1.0 + jnp.exp(-logits))


def _tc_head(xv2d, xw2d, w0, b0, w1, b1, w2, bias_comb, bs=512):
    nb = _B // bs
    return pl.pallas_call(
        _tc_body,
        grid=(nb,),
        in_specs=[
            pl.BlockSpec((bs, _F), lambda i: (i, 0)),
            pl.BlockSpec((bs, _F * _D), lambda i: (i, 0)),
            pl.BlockSpec((_F * _D, _H0), lambda i: (0, 0)),
            pl.BlockSpec((1, _H0), lambda i: (0, 0)),
            pl.BlockSpec((_H0, _H1), lambda i: (0, 0)),
            pl.BlockSpec((1, _H1), lambda i: (0, 0)),
            pl.BlockSpec((_H1, 1), lambda i: (0, 0)),
            pl.BlockSpec(memory_space=pltpu.SMEM),
        ],
        out_specs=pl.BlockSpec((bs,), lambda i: (i,)),
        out_shape=jax.ShapeDtypeStruct((_B,), jnp.float32),
    )(xw2d, xv2d, w0, b0, w1, b1, w2, bias_comb)


def kernel(indices, embed_tables, weight_tables, bias, w0, b0, w1, b1, w2, b2):
    idx = indices.astype(jnp.int32)
    flat_idx = (idx + (jnp.arange(_F, dtype=jnp.int32) * _V)[None, :])
    flat_idx = flat_idx.reshape(_NW, _NCH, _CHUNK)
    etab = embed_tables.reshape(_F * _V, _D)
    wtab16 = weight_tables.reshape(_F * _V // 16, 16)
    xv, xw = _sc_gather(flat_idx, etab, wtab16)
    xv2d = xv.reshape(_B, _F * _D)
    xw2d = xw.reshape(_B, _F)
    bias_comb = (bias[0] + b2[0]).reshape(1)
    return _tc_head(xv2d, xw2d, w0, b0.reshape(1, _H0), w1, b1.reshape(1, _H1),
                    w2, bias_comb)
